# baseline (speedup 1.0000x reference)
import functools
import math

import jax
import jax.numpy as jnp
from jax import lax
from jax.experimental import pallas as pl
from jax.experimental.pallas import tpu as pltpu
from jax.experimental.pallas import tpu_sc as plsc

B = 4096
IN_DIM = 768
SIG_DIM = 32
E = 8
EXPERT_DIM = 768
HID = E * 2

KA = IN_DIM + 128
TB = 512
NB = B // TB
G = NB + E - 1

NC = 2
NS = 16
NW = NC * NS
RPW = B // NW

_SQRT2 = math.sqrt(2.0)


def _router_kernel(x_ref, s2_ref, w1x_ref, w1s_ref, b1_ref, w2_ref, b2_ref,
                   xaug_ref, idx_ref):
    xb = x_ref[...]
    h = (jnp.dot(xb, w1x_ref[...], preferred_element_type=jnp.float32)
         + jnp.dot(s2_ref[...], w1s_ref[...], preferred_element_type=jnp.float32)
         + b1_ref[...])
    h = 0.5 * h * (1.0 + jax.lax.erf(h / _SQRT2))
    logits = jnp.dot(h, w2_ref[...], preferred_element_type=jnp.float32) + b2_ref[...]
    m = jnp.max(logits, axis=-1, keepdims=True)
    ssum = jnp.sum(jnp.exp(logits - m), axis=-1, keepdims=True)
    w = 1.0 / ssum
    idx = jnp.argmax(logits, axis=-1)[:, None]

    xaug_ref[:, :IN_DIM] = w * xb
    lane = lax.broadcasted_iota(jnp.int32, (TB, KA - IN_DIM), 1)
    xaug_ref[:, IN_DIM:] = jnp.where(lane == 0, w, 0.0)
    idx_ref[...] = idx


def _run_router(x, s2, w1x, w1s, b1, W2, b2):
    return pl.pallas_call(
        _router_kernel,
        grid=(NB,),
        in_specs=[
            pl.BlockSpec((TB, IN_DIM), lambda i: (i, 0)),
            pl.BlockSpec((TB, SIG_DIM + 3), lambda i: (i, 0)),
            pl.BlockSpec(w1x.shape, lambda i: (0, 0)),
            pl.BlockSpec(w1s.shape, lambda i: (0, 0)),
            pl.BlockSpec((1, HID), lambda i: (0, 0)),
            pl.BlockSpec(W2.shape, lambda i: (0, 0)),
            pl.BlockSpec((1, E), lambda i: (0, 0)),
        ],
        out_specs=[
            pl.BlockSpec((TB, KA), lambda i: (i, 0)),
            pl.BlockSpec((TB, 1), lambda i: (i, 0)),
        ],
        out_shape=[
            jax.ShapeDtypeStruct((B, KA), jnp.float32),
            jax.ShapeDtypeStruct((B, 1), jnp.int32),
        ],
    )(x, s2, w1x, w1s, b1, W2, b2)


def _permute_rows(src, idxs, d):
    mesh = plsc.VectorSubcoreMesh(core_axis_name="c", subcore_axis_name="s")

    @functools.partial(
        pl.kernel,
        out_type=jax.ShapeDtypeStruct((B, d), jnp.float32),
        mesh=mesh,
        scratch_types=[
            pltpu.VMEM((RPW,), jnp.int32),
            pltpu.VMEM((RPW, d), jnp.float32),
            pltpu.SemaphoreType.DMA,
        ],
    )
    def _perm(src_hbm, idx_hbm, out_hbm, idx_v, rows_v, sem):
        wid = lax.axis_index("s") * NC + lax.axis_index("c")
        base = wid * RPW
        pltpu.sync_copy(idx_hbm.at[pl.ds(base, RPW)], idx_v)
        pltpu.async_copy(src_hbm.at[idx_v], rows_v, sem).wait()
        pltpu.sync_copy(rows_v, out_hbm.at[pl.ds(base, RPW)])

    return _perm(src, idxs)


def _grouped_kernel(blk_ref, we_ref_idx, first_ref, off_ref, es_ref,
                    xs_ref, we_ref, out_ref):
    g = pl.program_id(0)
    b = blk_ref[g]
    e = es_ref[g]
    lo = off_ref[e]
    hi = off_ref[e + 1]
    rowpos = b * TB + lax.broadcasted_iota(jnp.int32, (TB, 1), 0)
    mask = ((rowpos >= lo) & (rowpos < hi)).astype(jnp.float32)
    prod = jnp.dot(xs_ref[...], we_ref[0], preferred_element_type=jnp.float32)
    contrib = mask * prod

    @pl.when(first_ref[g] == 1)
    def _():
        out_ref[...] = contrib

    @pl.when(first_ref[g] == 0)
    def _():
        out_ref[...] = out_ref[...] + contrib


def _run_grouped(blk, wi, first, off_ext, es, xsorted, we_aug):
    grid_spec = pltpu.PrefetchScalarGridSpec(
        num_scalar_prefetch=5,
        grid=(G,),
        in_specs=[
            pl.BlockSpec((TB, KA), lambda g, blk, wi, fi, off, es: (blk[g], 0)),
            pl.BlockSpec((1, KA, EXPERT_DIM),
                         lambda g, blk, wi, fi, off, es: (wi[g], 0, 0)),
        ],
        out_specs=pl.BlockSpec((TB, EXPERT_DIM),
                               lambda g, blk, wi, fi, off, es: (blk[g], 0)),
    )
    return pl.pallas_call(
        _grouped_kernel,
        grid_spec=grid_spec,
        out_shape=jax.ShapeDtypeStruct((B, EXPERT_DIM), jnp.float32),
    )(blk, wi, first, off_ext, es, xsorted, we_aug)


@jax.jit
def kernel(x, physical_signature, task_context, resource_state,
           W1, b1, W2, b2, We, be):
    s2 = jnp.concatenate([physical_signature, task_context, resource_state],
                         axis=-1)
    w1x = W1[:IN_DIM]
    w1s = W1[IN_DIM:]

    xaug, idxo = _run_router(x, s2, w1x, w1s, b1[None, :], W2, b2[None, :])
    idx = idxo[:, 0]

    i32 = jnp.int32
    oh = (idx[:, None] == jnp.arange(E, dtype=i32)[None, :]).astype(i32)
    c = jnp.cumsum(oh, axis=0)
    counts = c[-1]
    ends = jnp.cumsum(counts)
    off = jnp.concatenate([jnp.zeros((1,), i32), ends]).astype(i32)
    off_ext = jnp.concatenate([off, jnp.full((1,), B, i32)])
    rank = jnp.take_along_axis(c, idx[:, None], axis=1)[:, 0] - 1
    position = off[idx] + rank
    order = jnp.zeros((B,), i32).at[position].set(jnp.arange(B, dtype=i32))

    bb = jnp.arange(NB, dtype=i32)
    e_lo = jnp.searchsorted(ends, bb * TB, side="right").astype(i32)
    e_hi = jnp.searchsorted(ends, (bb + 1) * TB - 1, side="right").astype(i32)
    spans = e_hi - e_lo + 1
    start = jnp.concatenate([jnp.zeros((1,), i32),
                             jnp.cumsum(spans)]).astype(i32)
    g = jnp.arange(G, dtype=i32)
    b_of_g = jnp.clip(jnp.searchsorted(start, g, side="right") - 1,
                      0, NB - 1).astype(i32)
    e_of_g = jnp.clip(e_lo[b_of_g] + (g - start[b_of_g]), 0, E).astype(i32)
    first_of_g = (g == start[b_of_g]).astype(i32)
    we_of_g = jnp.minimum(e_of_g, E - 1)

    xsorted = _permute_rows(xaug, order, KA)
    we_aug = jnp.concatenate(
        [We, be[:, None, :], jnp.zeros((E, KA - IN_DIM - 1, EXPERT_DIM),
                                       jnp.float32)], axis=1)
    sortedraw = _run_grouped(b_of_g, we_of_g, first_of_g, off_ext, e_of_g,
                             xsorted, we_aug)
    out = _permute_rows(sortedraw, position, EXPERT_DIM)
    return out


# speedup vs baseline: 1.9890x; 1.9890x over previous
import math

import jax
import jax.numpy as jnp
from jax import lax
from jax.experimental import pallas as pl

B = 4096
IN_DIM = 768
SIG_DIM = 32
E = 8
EXPERT_DIM = 768
HID = E * 2

TB = 512
NB = B // TB

_SQRT2 = math.sqrt(2.0)


def _moe_kernel(x_ref, s2_ref, w1x_ref, w1s_ref, b1_ref, w2_ref, b2_ref,
                we_ref, be_ref, out_ref):
    xb = x_ref[...]
    h = (jnp.dot(xb, w1x_ref[...], preferred_element_type=jnp.float32)
         + jnp.dot(s2_ref[...], w1s_ref[...], preferred_element_type=jnp.float32)
         + b1_ref[...])
    h = 0.5 * h * (1.0 + jax.lax.erf(h / _SQRT2))
    logits = jnp.dot(h, w2_ref[...], preferred_element_type=jnp.float32) + b2_ref[...]
    m = jnp.max(logits, axis=-1, keepdims=True)
    ssum = jnp.sum(jnp.exp(logits - m), axis=-1, keepdims=True)
    w = 1.0 / ssum
    idx = jnp.argmax(logits, axis=-1)[:, None]

    xb16 = xb.astype(jnp.bfloat16)
    zero16 = jnp.zeros((TB, IN_DIM), jnp.bfloat16)
    pieces = []
    wjs = []
    for j in range(E):
        sel = idx == j
        wjs.append(jnp.where(sel, w, 0.0))
        pieces.append(jnp.where(sel, xb16, zero16))
    xbig = jnp.concatenate(pieces, axis=1)
    ex = lax.dot_general(xbig, we_ref[...], (((1,), (0,)), ((), ())),
                         precision=lax.Precision.DEFAULT,
                         preferred_element_type=jnp.float32)
    wj8 = jnp.concatenate(wjs, axis=1)
    bias = jnp.dot(wj8, be_ref[...], preferred_element_type=jnp.float32)
    out_ref[...] = w * ex + bias


@jax.jit
def kernel(x, physical_signature, task_context, resource_state,
           W1, b1, W2, b2, We, be):
    s2 = jnp.concatenate([physical_signature, task_context, resource_state],
                         axis=-1)
    w1x = W1[:IN_DIM]
    w1s = W1[IN_DIM:]
    we_flat = We.astype(jnp.bfloat16).reshape(E * IN_DIM, EXPERT_DIM)

    grid = (NB,)
    out = pl.pallas_call(
        _moe_kernel,
        grid=grid,
        in_specs=[
            pl.BlockSpec((TB, IN_DIM), lambda i: (i, 0)),
            pl.BlockSpec((TB, SIG_DIM + 3), lambda i: (i, 0)),
            pl.BlockSpec(w1x.shape, lambda i: (0, 0)),
            pl.BlockSpec(w1s.shape, lambda i: (0, 0)),
            pl.BlockSpec((1, HID), lambda i: (0, 0)),
            pl.BlockSpec(W2.shape, lambda i: (0, 0)),
            pl.BlockSpec((1, E), lambda i: (0, 0)),
            pl.BlockSpec((E * IN_DIM, EXPERT_DIM), lambda i: (0, 0)),
            pl.BlockSpec(be.shape, lambda i: (0, 0)),
        ],
        out_specs=pl.BlockSpec((TB, EXPERT_DIM), lambda i: (i, 0)),
        out_shape=jax.ShapeDtypeStruct((B, EXPERT_DIM), jnp.float32),
    )(x, s2, w1x, w1s, b1[None, :], W2, b2[None, :], we_flat, be)
    return out
